# baseline (speedup 1.0000x reference)
import jax
import jax.numpy as jnp
from jax import lax
from jax.experimental import pallas as pl
from jax.experimental.pallas import tpu as pltpu
from jax.experimental.pallas import tpu_sc as plsc

N_TOKENS = 32768
INPUT_DIM = 768
NUM_EXPERTS = 8
NC, NS, L = 2, 16, 16
TOK_PER_CORE = N_TOKENS // NC
CHUNK = 32
G = CHUNK // L
SUP = NS * CHUNK
N_SUP = TOK_PER_CORE // SUP


def _sc_body(x_hbm, w_hbm, b_hbm, o_hbm,
             sp0, sp1, xbuf, wv, bv, obuf, sem):
    cid = lax.axis_index("c")
    sid = lax.axis_index("s")
    core_base = cid * TOK_PER_CORE

    pltpu.sync_copy(w_hbm, wv)
    pltpu.sync_copy(b_hbm, bv)

    lane = lax.iota(jnp.int32, L)
    rows = [lane + (g * L) for g in range(G)]
    e_idx = [jnp.full((L,), e, jnp.int32) for e in range(NUM_EXPERTS)]
    bbc = [bv[e] for e in range(NUM_EXPERTS)]

    sbufs = [sp0, sp1]

    def src(c):
        return x_hbm.at[pl.ds(core_base + c * SUP, SUP)]

    for c in range(N_SUP):
        @pl.when(sid == 0)
        def _(c=c):
            if c == 0:
                pltpu.make_async_copy(src(0), sbufs[0], sem).start()
            pltpu.make_async_copy(src(c), sbufs[c % 2], sem).wait()
            if c + 1 < N_SUP:
                pltpu.make_async_copy(
                    src(c + 1), sbufs[(c + 1) % 2], sem).start()

        plsc.subcore_barrier()
        pltpu.sync_copy(sbufs[c % 2].at[pl.ds(sid * CHUNK, CHUNK)], xbuf)

        def dbody(d, accs):
            accs = list(accs)
            dcol = jnp.full((L,), d, jnp.int32)
            xv = [plsc.load_gather(xbuf, [rows[g], dcol]) for g in range(G)]
            for e in range(NUM_EXPERTS):
                wbc = plsc.load_gather(wv, [dcol, e_idx[e]])
                for g in range(G):
                    k = g * NUM_EXPERTS + e
                    accs[k] = accs[k] + xv[g] * wbc
            return tuple(accs)

        init = tuple(bbc[e] for g in range(G) for e in range(NUM_EXPERTS))
        accs = lax.fori_loop(0, INPUT_DIM, dbody, init)

        for g in range(G):
            a = [accs[g * NUM_EXPERTS + e] for e in range(NUM_EXPERTS)]
            m = a[0]
            for e in range(1, NUM_EXPERTS):
                m = jnp.maximum(m, a[e])
            ex = [jnp.exp(v - m) for v in a]
            s = ex[0]
            for e in range(1, NUM_EXPERTS):
                s = s + ex[e]
            r = 1.0 / s
            for e in range(NUM_EXPERTS):
                plsc.store_scatter(obuf, [rows[g], e_idx[e]], ex[e] * r)

        pltpu.sync_copy(
            obuf,
            o_hbm.at[pl.ds(core_base + c * SUP + sid * CHUNK, CHUNK)])
        plsc.subcore_barrier()


def kernel(x, W, b):
    b2 = jnp.tile(b.reshape(NUM_EXPERTS, 1), (1, L))
    mesh = plsc.VectorSubcoreMesh(core_axis_name="c", subcore_axis_name="s")
    f = pl.kernel(
        _sc_body,
        out_type=jax.ShapeDtypeStruct((N_TOKENS, NUM_EXPERTS), jnp.float32),
        mesh=mesh,
        scratch_types=[
            pltpu.VMEM_SHARED((SUP, INPUT_DIM), jnp.float32),
            pltpu.VMEM_SHARED((SUP, INPUT_DIM), jnp.float32),
            pltpu.VMEM((CHUNK, INPUT_DIM), jnp.float32),
            pltpu.VMEM((INPUT_DIM, NUM_EXPERTS), jnp.float32),
            pltpu.VMEM((NUM_EXPERTS, L), jnp.float32),
            pltpu.VMEM((CHUNK, NUM_EXPERTS), jnp.float32),
            pltpu.SemaphoreType.DMA,
        ],
        compiler_params=pltpu.CompilerParams(
            needs_layout_passes=False, use_tc_tiling_on_sc=False),
    )
    return f(x, W, b2)


# speedup vs baseline: 1.6787x; 1.6787x over previous
import jax
import jax.numpy as jnp
from jax import lax
from jax.experimental import pallas as pl
from jax.experimental.pallas import tpu as pltpu
from jax.experimental.pallas import tpu_sc as plsc

N_TOKENS = 32768
INPUT_DIM = 768
NUM_EXPERTS = 8
NC, NS, L = 2, 16, 16
TOK_PER_CORE = N_TOKENS // NC
CHUNK = 32
G = CHUNK // L
SUP = NS * CHUNK
N_SUP = TOK_PER_CORE // SUP


def _sc_body(x_hbm, w_hbm, b_hbm, o_hbm,
             sp0, sp1, xbuf, wv, bv, obuf, sem):
    cid = lax.axis_index("c")
    sid = lax.axis_index("s")
    core_base = cid * TOK_PER_CORE

    pltpu.sync_copy(w_hbm, wv)
    pltpu.sync_copy(b_hbm, bv)

    lane = lax.iota(jnp.int32, L)
    rows = [lane + (g * L) for g in range(G)]
    e_idx = [jnp.full((L,), e, jnp.int32) for e in range(NUM_EXPERTS)]
    bbc = [bv[e] for e in range(NUM_EXPERTS)]

    sbufs = [sp0, sp1]

    def src(c):
        return x_hbm.at[pl.ds(core_base + c * SUP, SUP)]

    for c in range(N_SUP):
        @pl.when(sid == 0)
        def _(c=c):
            if c == 0:
                pltpu.make_async_copy(src(0), sbufs[0], sem).start()
            pltpu.make_async_copy(src(c), sbufs[c % 2], sem).wait()
            if c + 1 < N_SUP:
                pltpu.make_async_copy(
                    src(c + 1), sbufs[(c + 1) % 2], sem).start()

        plsc.subcore_barrier()
        pltpu.sync_copy(sbufs[c % 2].at[pl.ds(sid * CHUNK, CHUNK)], xbuf)

        def dbody(d, accs):
            accs = list(accs)
            dsk = lane + d
            dsk = jnp.where(dsk >= INPUT_DIM, dsk - INPUT_DIM, dsk)
            xv = [plsc.load_gather(xbuf, [rows[g], dsk]) for g in range(G)]
            for e in range(NUM_EXPERTS):
                wbc = plsc.load_gather(wv, [dsk, e_idx[e]])
                for g in range(G):
                    k = g * NUM_EXPERTS + e
                    accs[k] = accs[k] + xv[g] * wbc
            return tuple(accs)

        init = tuple(bbc[e] for g in range(G) for e in range(NUM_EXPERTS))
        accs = lax.fori_loop(0, INPUT_DIM, dbody, init)

        for g in range(G):
            a = [accs[g * NUM_EXPERTS + e] for e in range(NUM_EXPERTS)]
            m = a[0]
            for e in range(1, NUM_EXPERTS):
                m = jnp.maximum(m, a[e])
            ex = [jnp.exp(v - m) for v in a]
            s = ex[0]
            for e in range(1, NUM_EXPERTS):
                s = s + ex[e]
            r = 1.0 / s
            for e in range(NUM_EXPERTS):
                plsc.store_scatter(obuf, [rows[g], e_idx[e]], ex[e] * r)

        pltpu.sync_copy(
            obuf,
            o_hbm.at[pl.ds(core_base + c * SUP + sid * CHUNK, CHUNK)])
        plsc.subcore_barrier()


def kernel(x, W, b):
    b2 = jnp.tile(b.reshape(NUM_EXPERTS, 1), (1, L))
    W17 = jnp.concatenate(
        [W, jnp.zeros((INPUT_DIM, 17 - NUM_EXPERTS), jnp.float32)], axis=1)
    mesh = plsc.VectorSubcoreMesh(core_axis_name="c", subcore_axis_name="s")
    f = pl.kernel(
        _sc_body,
        out_type=jax.ShapeDtypeStruct((N_TOKENS, NUM_EXPERTS), jnp.float32),
        mesh=mesh,
        scratch_types=[
            pltpu.VMEM_SHARED((SUP, INPUT_DIM), jnp.float32),
            pltpu.VMEM_SHARED((SUP, INPUT_DIM), jnp.float32),
            pltpu.VMEM((CHUNK, INPUT_DIM), jnp.float32),
            pltpu.VMEM((INPUT_DIM, 17), jnp.float32),
            pltpu.VMEM((NUM_EXPERTS, L), jnp.float32),
            pltpu.VMEM((CHUNK, NUM_EXPERTS), jnp.float32),
            pltpu.SemaphoreType.DMA,
        ],
        compiler_params=pltpu.CompilerParams(
            needs_layout_passes=False, use_tc_tiling_on_sc=False),
    )
    return f(x, W17, b2)


# speedup vs baseline: 1.6791x; 1.0002x over previous
import jax
import jax.numpy as jnp
from jax import lax
from jax.experimental import pallas as pl
from jax.experimental.pallas import tpu as pltpu
from jax.experimental.pallas import tpu_sc as plsc

N_TOKENS = 32768
INPUT_DIM = 768
NUM_EXPERTS = 8
NC, NS, L = 2, 16, 16
TOK_PER_CORE = N_TOKENS // NC
CHUNK = 32
G = CHUNK // L
SUP = NS * CHUNK
N_SUP = TOK_PER_CORE // SUP


def _sc_body(x_hbm, w_hbm, b_hbm, o_hbm,
             sp0, sp1, xbuf, wv, bv, obuf, sem):
    cid = lax.axis_index("c")
    sid = lax.axis_index("s")
    core_base = cid * TOK_PER_CORE

    pltpu.sync_copy(w_hbm, wv)
    pltpu.sync_copy(b_hbm, bv)

    lane = lax.iota(jnp.int32, L)
    rows = [lane + (g * L) for g in range(G)]
    e_idx = [jnp.full((L,), e, jnp.int32) for e in range(NUM_EXPERTS)]
    bbc = [bv[e] for e in range(NUM_EXPERTS)]

    sbufs = [sp0, sp1]

    def src(c):
        return x_hbm.at[pl.ds(core_base + c * SUP, SUP)]

    for c in range(N_SUP):
        @pl.when(sid == 0)
        def _(c=c):
            if c == 0:
                pltpu.make_async_copy(src(0), sbufs[0], sem).start()
            pltpu.make_async_copy(src(c), sbufs[c % 2], sem).wait()
            if c + 1 < N_SUP:
                pltpu.make_async_copy(
                    src(c + 1), sbufs[(c + 1) % 2], sem).start()

        plsc.subcore_barrier()
        pltpu.sync_copy(sbufs[c % 2].at[pl.ds(sid * CHUNK, CHUNK)], xbuf)

        init = tuple(bbc[e] for g in range(G) for e in range(NUM_EXPERTS))

        @plsc.parallel_loop(0, INPUT_DIM, unroll=2, carry=init)
        def accs(d, accs):
            accs = list(accs)
            dsk = lane + d
            dsk = jnp.where(dsk >= INPUT_DIM, dsk - INPUT_DIM, dsk)
            xv = [plsc.load_gather(xbuf, [rows[g], dsk]) for g in range(G)]
            for e in range(NUM_EXPERTS):
                wbc = plsc.load_gather(wv, [dsk, e_idx[e]])
                for g in range(G):
                    k = g * NUM_EXPERTS + e
                    accs[k] = accs[k] + xv[g] * wbc
            return tuple(accs)

        for g in range(G):
            a = [accs[g * NUM_EXPERTS + e] for e in range(NUM_EXPERTS)]
            m = a[0]
            for e in range(1, NUM_EXPERTS):
                m = jnp.maximum(m, a[e])
            ex = [jnp.exp(v - m) for v in a]
            s = ex[0]
            for e in range(1, NUM_EXPERTS):
                s = s + ex[e]
            r = 1.0 / s
            for e in range(NUM_EXPERTS):
                plsc.store_scatter(obuf, [rows[g], e_idx[e]], ex[e] * r)

        pltpu.sync_copy(
            obuf,
            o_hbm.at[pl.ds(core_base + c * SUP + sid * CHUNK, CHUNK)])
        plsc.subcore_barrier()


def kernel(x, W, b):
    b2 = jnp.tile(b.reshape(NUM_EXPERTS, 1), (1, L))
    W17 = jnp.concatenate(
        [W, jnp.zeros((INPUT_DIM, 17 - NUM_EXPERTS), jnp.float32)], axis=1)
    mesh = plsc.VectorSubcoreMesh(core_axis_name="c", subcore_axis_name="s")
    f = pl.kernel(
        _sc_body,
        out_type=jax.ShapeDtypeStruct((N_TOKENS, NUM_EXPERTS), jnp.float32),
        mesh=mesh,
        scratch_types=[
            pltpu.VMEM_SHARED((SUP, INPUT_DIM), jnp.float32),
            pltpu.VMEM_SHARED((SUP, INPUT_DIM), jnp.float32),
            pltpu.VMEM((CHUNK, INPUT_DIM), jnp.float32),
            pltpu.VMEM((INPUT_DIM, 17), jnp.float32),
            pltpu.VMEM((NUM_EXPERTS, L), jnp.float32),
            pltpu.VMEM((CHUNK, NUM_EXPERTS), jnp.float32),
            pltpu.SemaphoreType.DMA,
        ],
        compiler_params=pltpu.CompilerParams(
            needs_layout_passes=False, use_tc_tiling_on_sc=False),
    )
    return f(x, W17, b2)


# speedup vs baseline: 5.0079x; 2.9825x over previous
import jax
import jax.numpy as jnp
from jax import lax
from jax.experimental import pallas as pl
from jax.experimental.pallas import tpu as pltpu
from jax.experimental.pallas import tpu_sc as plsc

N_TOKENS = 32768
INPUT_DIM = 768
NUM_EXPERTS = 8

SC_TOKENS = 2048
TC_TOKENS = N_TOKENS - SC_TOKENS

BLK_T = 2048


def _tc_body(x_ref, w_ref, b_ref, o_ref):
    xb = x_ref[...].astype(jnp.bfloat16)
    wb = w_ref[...].astype(jnp.bfloat16)
    logits = jnp.dot(xb, wb, preferred_element_type=jnp.float32) + b_ref[...]
    m = jnp.max(logits, axis=-1, keepdims=True)
    e = jnp.exp(logits - m)
    s = jnp.sum(e, axis=-1, keepdims=True)
    o_ref[...] = (e / s).T


def _tc_part(x, W, b):
    b2 = b.reshape(1, NUM_EXPERTS)
    grid = (TC_TOKENS // BLK_T,)
    out_t = pl.pallas_call(
        _tc_body,
        grid=grid,
        in_specs=[
            pl.BlockSpec((BLK_T, INPUT_DIM), lambda i: (i, 0)),
            pl.BlockSpec((INPUT_DIM, NUM_EXPERTS), lambda i: (0, 0)),
            pl.BlockSpec((1, NUM_EXPERTS), lambda i: (0, 0)),
        ],
        out_specs=pl.BlockSpec((NUM_EXPERTS, BLK_T), lambda i: (0, i)),
        out_shape=jax.ShapeDtypeStruct((NUM_EXPERTS, TC_TOKENS), jnp.float32),
    )(x, W, b2)
    return out_t.T


NC, NS, L = 2, 16, 16
TOK_PER_CORE = SC_TOKENS // NC
CHUNK = 32
G = CHUNK // L
SUP = NS * CHUNK
N_SUP = TOK_PER_CORE // SUP


def _sc_body(x_hbm, w_hbm, b_hbm, o_hbm,
             sp0, sp1, xbuf, wv, bv, obuf, sem):
    cid = lax.axis_index("c")
    sid = lax.axis_index("s")
    core_base = cid * TOK_PER_CORE

    pltpu.sync_copy(w_hbm, wv)
    pltpu.sync_copy(b_hbm, bv)

    lane = lax.iota(jnp.int32, L)
    rows = [lane + (g * L) for g in range(G)]
    e_idx = [jnp.full((L,), e, jnp.int32) for e in range(NUM_EXPERTS)]
    bbc = [bv[e] for e in range(NUM_EXPERTS)]

    sbufs = [sp0, sp1]

    def src(c):
        return x_hbm.at[pl.ds(TC_TOKENS + core_base + c * SUP, SUP)]

    for c in range(N_SUP):
        @pl.when(sid == 0)
        def _(c=c):
            if c == 0:
                pltpu.make_async_copy(src(0), sbufs[0], sem).start()
            pltpu.make_async_copy(src(c), sbufs[c % 2], sem).wait()
            if c + 1 < N_SUP:
                pltpu.make_async_copy(
                    src(c + 1), sbufs[(c + 1) % 2], sem).start()

        plsc.subcore_barrier()
        pltpu.sync_copy(sbufs[c % 2].at[pl.ds(sid * CHUNK, CHUNK)], xbuf)

        init = tuple(bbc[e] for g in range(G) for e in range(NUM_EXPERTS))

        @plsc.parallel_loop(0, INPUT_DIM, unroll=2, carry=init)
        def accs(d, accs):
            accs = list(accs)
            dsk = lane + d
            dsk = jnp.where(dsk >= INPUT_DIM, dsk - INPUT_DIM, dsk)
            xv = [plsc.load_gather(xbuf, [rows[g], dsk]) for g in range(G)]
            for e in range(NUM_EXPERTS):
                wbc = plsc.load_gather(wv, [dsk, e_idx[e]])
                for g in range(G):
                    k = g * NUM_EXPERTS + e
                    accs[k] = accs[k] + xv[g] * wbc
            return tuple(accs)

        for g in range(G):
            a = [accs[g * NUM_EXPERTS + e] for e in range(NUM_EXPERTS)]
            m = a[0]
            for e in range(1, NUM_EXPERTS):
                m = jnp.maximum(m, a[e])
            ex = [jnp.exp(v - m) for v in a]
            s = ex[0]
            for e in range(1, NUM_EXPERTS):
                s = s + ex[e]
            r = 1.0 / s
            for e in range(NUM_EXPERTS):
                plsc.store_scatter(obuf, [rows[g], e_idx[e]], ex[e] * r)

        pltpu.sync_copy(
            obuf,
            o_hbm.at[pl.ds(core_base + c * SUP + sid * CHUNK, CHUNK)])
        plsc.subcore_barrier()


def _sc_part(x, W, b):
    b2 = jnp.tile(b.reshape(NUM_EXPERTS, 1), (1, L))
    W17 = jnp.concatenate(
        [W, jnp.zeros((INPUT_DIM, 17 - NUM_EXPERTS), jnp.float32)], axis=1)
    mesh = plsc.VectorSubcoreMesh(core_axis_name="c", subcore_axis_name="s")
    f = pl.kernel(
        _sc_body,
        out_type=jax.ShapeDtypeStruct((SC_TOKENS, NUM_EXPERTS), jnp.float32),
        mesh=mesh,
        scratch_types=[
            pltpu.VMEM_SHARED((SUP, INPUT_DIM), jnp.float32),
            pltpu.VMEM_SHARED((SUP, INPUT_DIM), jnp.float32),
            pltpu.VMEM((CHUNK, INPUT_DIM), jnp.float32),
            pltpu.VMEM((INPUT_DIM, 17), jnp.float32),
            pltpu.VMEM((NUM_EXPERTS, L), jnp.float32),
            pltpu.VMEM((CHUNK, NUM_EXPERTS), jnp.float32),
            pltpu.SemaphoreType.DMA,
        ],
        compiler_params=pltpu.CompilerParams(
            needs_layout_passes=False, use_tc_tiling_on_sc=False),
    )
    return f(x, W17, b2)


def kernel(x, W, b):
    out_sc = _sc_part(x, W, b)
    out_tc = _tc_part(x, W, b)
    return jnp.concatenate([out_tc, out_sc], axis=0)


# speedup vs baseline: 21.5784x; 4.3089x over previous
import jax
import jax.numpy as jnp
from jax.experimental import pallas as pl
from jax.experimental.pallas import tpu as pltpu

N_TOKENS = 32768
INPUT_DIM = 768
NUM_EXPERTS = 8
BLK_T = 4096


def _router_body(x_ref, w_ref, b_ref, o_ref):
    xb = x_ref[...].astype(jnp.bfloat16)
    wb = w_ref[...].astype(jnp.bfloat16)
    logits = jnp.dot(xb, wb, preferred_element_type=jnp.float32) + b_ref[...]
    m = jnp.max(logits, axis=-1, keepdims=True)
    e = jnp.exp(logits - m)
    s = jnp.sum(e, axis=-1, keepdims=True)
    o_ref[...] = (e / s).T


def kernel(x, W, b):
    b2 = b.reshape(1, NUM_EXPERTS)
    grid = (N_TOKENS // BLK_T,)
    out_t = pl.pallas_call(
        _router_body,
        grid=grid,
        in_specs=[
            pl.BlockSpec((BLK_T, INPUT_DIM), lambda i: (i, 0)),
            pl.BlockSpec((INPUT_DIM, NUM_EXPERTS), lambda i: (0, 0)),
            pl.BlockSpec((1, NUM_EXPERTS), lambda i: (0, 0)),
        ],
        out_specs=pl.BlockSpec((NUM_EXPERTS, BLK_T), lambda i: (0, i)),
        out_shape=jax.ShapeDtypeStruct((NUM_EXPERTS, N_TOKENS), jnp.float32),
    )(x, W, b2)
    return out_t.T
